# baseline (speedup 1.0000x reference)
import functools

import jax
import jax.numpy as jnp
from jax import lax
from jax.experimental import pallas as pl
from jax.experimental.pallas import tpu as pltpu
from jax.experimental.pallas import tpu_sc as plsc

_N = 256
_E = 4096
_NC = 2
_NS = 16
_EPT = _E // _NS
_LANES = 16


def _pre_body(x_e, w1_e, av1_e, ea_e, we1_e, ae1_e, we2_e, ae2_e,
              x_g, w1_g, av1_g, ea_g, we1_g, ae1_g, we2_g, ae2_g,
              h1_o, sd1_o, esc_o):
    for b, (x, w1, av, ea, we1, ae1, we2, ae2) in enumerate((
            (x_e, w1_e, av1_e, ea_e, we1_e, ae1_e, we2_e, ae2_e),
            (x_g, w1_g, av1_g, ea_g, we1_g, ae1_g, we2_g, ae2_g))):
        h = jnp.dot(x[...], w1[...], preferred_element_type=jnp.float32)
        h1_o[b] = h
        sd1_o[b] = jnp.dot(h, av[...], preferred_element_type=jnp.float32)
        wc1 = jnp.dot(we1[...], ae1[...],
                      preferred_element_type=jnp.float32)
        wc2 = jnp.dot(we2[...], ae2[...],
                      preferred_element_type=jnp.float32)
        esc_o[b] = jnp.dot(ea[...], jnp.concatenate([wc1, wc2], axis=1),
                           preferred_element_type=jnp.float32)


def _tc_pre(*args):
    return pl.pallas_call(
        _pre_body,
        out_shape=[
            jax.ShapeDtypeStruct((2, _N, 512), jnp.float32),
            jax.ShapeDtypeStruct((2, _N, 2), jnp.float32),
            jax.ShapeDtypeStruct((2, _E, 2), jnp.float32),
        ],
    )(*args)


def _sc_edge(sd, esc, ei, zeros, layer):
    mesh = plsc.VectorSubcoreMesh(
        core_axis_name="c", subcore_axis_name="s",
        num_cores=_NC, num_subcores=_NS)

    @functools.partial(
        pl.kernel,
        out_type=jax.ShapeDtypeStruct((2, _N * _N), jnp.float32),
        mesh=mesh,
        scratch_types=[
            pltpu.VMEM((_N, 2), jnp.float32),
            pltpu.VMEM((_EPT, 2), jnp.float32),
            pltpu.VMEM((_EPT,), jnp.int32),
            pltpu.VMEM((_EPT,), jnp.int32),
            pltpu.VMEM((2, 128), jnp.float32),
            pltpu.VMEM((2, 128), jnp.int32),
            pltpu.VMEM_SHARED((_N * _N,), jnp.float32),
        ],
        compiler_params=pltpu.CompilerParams(needs_layout_passes=False),
    )
    def k(sd_h, esc_h, ei_h, zeros_h, a_out,
          sd_v, esc_v, src_v, dst_v, ex_v, idx_v, a_sh):
        c = lax.axis_index("c")
        s = lax.axis_index("s")
        base = s * _EPT
        pltpu.sync_copy(sd_h.at[c], sd_v)
        pltpu.sync_copy(esc_h.at[c, pl.ds(base, _EPT)], esc_v)
        pltpu.sync_copy(ei_h.at[c, 0, pl.ds(base, _EPT)], src_v)
        pltpu.sync_copy(ei_h.at[c, 1, pl.ds(base, _EPT)], dst_v)

        @pl.when(s == 0)
        def _():
            pltpu.sync_copy(zeros_h, a_sh)

        zero16 = jnp.zeros((_LANES,), jnp.int32)
        one16 = jnp.ones((_LANES,), jnp.int32)
        lay16 = jnp.full((_LANES,), layer, jnp.int32)
        for j in range(_EPT // _LANES):
            si = src_v[pl.ds(j * _LANES, _LANES)]
            di = dst_v[pl.ds(j * _LANES, _LANES)]
            sa = plsc.load_gather(sd_v, [si, zero16])
            sb = plsc.load_gather(sd_v, [di, one16])
            eidx = lax.iota(jnp.int32, _LANES) + j * _LANES
            ev = plsc.load_gather(esc_v, [eidx, lay16])
            t = sa + sb + ev
            t = jnp.maximum(t, t * jnp.float32(0.2))
            ex_v[j // 8, pl.ds((j % 8) * _LANES, _LANES)] = jnp.exp(t)
            idx_v[j // 8, pl.ds((j % 8) * _LANES, _LANES)] = di * _N + si

        plsc.subcore_barrier()
        for g in range(2):
            pltpu.sync_copy(ex_v.at[g], a_sh.at[idx_v.at[g]], add=True)
        plsc.subcore_barrier()

        @pl.when(s == 0)
        def _():
            pltpu.sync_copy(a_sh, a_out.at[c])

    return k(sd, esc, ei, zeros)


def _mid_body(a, h1, b1_e, b1_g, w2_e, w2_g, av2_e, av2_g,
              h2_o, sd2_o):
    for b, (b1, w2, av) in enumerate(((b1_e, w2_e, av2_e),
                                      (b1_g, w2_g, av2_g))):
        am = a[b]
        den = jnp.sum(am, axis=1, keepdims=True) + jnp.float32(1e-16)
        m = jnp.dot(am, h1[b], preferred_element_type=jnp.float32) / den
        t = jnp.maximum(m + b1[...], 0.0)
        h2 = jnp.dot(t, w2[...], preferred_element_type=jnp.float32)
        h2_o[b] = h2
        sd2_o[b] = jnp.dot(h2, av[...], preferred_element_type=jnp.float32)


def _tc_mid(*args):
    return pl.pallas_call(
        _mid_body,
        out_shape=[
            jax.ShapeDtypeStruct((2, _N, 128), jnp.float32),
            jax.ShapeDtypeStruct((2, _N, 2), jnp.float32),
        ],
    )(*args)


def _fin_body(a, h2, b2_e, wp_e, bp_e, b2_g, wp_g, bp_g,
              w1a, b1a, w1b, b1b, w2a, b2a, w2b, b2b, out_o):
    feats = []
    for b, (b2, wp, bp) in enumerate(((b2_e, wp_e, bp_e),
                                      (b2_g, wp_g, bp_g))):
        am = a[b]
        den = jnp.sum(am, axis=1, keepdims=True) + jnp.float32(1e-16)
        o = jnp.dot(am, h2[b], preferred_element_type=jnp.float32) / den
        o = o + b2[...]
        feats.append(jnp.dot(o, wp[...], preferred_element_type=jnp.float32)
                     + bp[...])
    z = jnp.concatenate(feats, axis=0)
    t = z + jnp.sum(z, axis=0, keepdims=True)
    t = jnp.maximum(jnp.dot(t, w1a[...], preferred_element_type=jnp.float32)
                    + b1a[...], 0.0)
    t = jnp.dot(t, w1b[...], preferred_element_type=jnp.float32) + b1b[...]
    t = jnp.maximum(t, 0.0)
    t = t + jnp.sum(t, axis=0, keepdims=True)
    t = jnp.maximum(jnp.dot(t, w2a[...], preferred_element_type=jnp.float32)
                    + b2a[...], 0.0)
    out_o[...] = (jnp.dot(t, w2b[...], preferred_element_type=jnp.float32)
                  + b2b[...])


def _tc_fin(*args):
    return pl.pallas_call(
        _fin_body,
        out_shape=jax.ShapeDtypeStruct((2 * _N, 128), jnp.float32),
    )(*args)


def kernel(emg_x, emg_edge_index, emg_edge_attr,
           eeg_x, eeg_edge_index, eeg_edge_attr, params):
    pe = params["emg_gat"]
    pg = params["eeg_gat"]
    gin = params["gin"]
    row = lambda v: v[None, :]
    col = lambda v: v[:, None]
    sdcols = lambda p, l: jnp.stack([p[f"as{l}"], p[f"ad{l}"]], axis=1)

    h1, sd1, esc = _tc_pre(
        emg_x, pe["W1"], sdcols(pe, 1), emg_edge_attr,
        pe["We1"], col(pe["ae1"]), pe["We2"], col(pe["ae2"]),
        eeg_x, pg["W1"], sdcols(pg, 1), eeg_edge_attr,
        pg["We1"], col(pg["ae1"]), pg["We2"], col(pg["ae2"]))

    zeros = jnp.zeros((_N * _N,), jnp.float32)
    ei = jnp.stack([emg_edge_index, eeg_edge_index])
    a1 = _sc_edge(sd1, esc, ei, zeros, 0)

    h2, sd2 = _tc_mid(
        a1.reshape(2, _N, _N), h1, row(pe["b1"]), row(pg["b1"]),
        pe["W2"], pg["W2"], sdcols(pe, 2), sdcols(pg, 2))

    a2 = _sc_edge(sd2, esc, ei, zeros, 1)

    prj_e = params["emg_proj"]
    prj_g = params["eeg_proj"]
    return _tc_fin(
        a2.reshape(2, _N, _N), h2,
        row(pe["b2"]), prj_e["W"], row(prj_e["b"]),
        row(pg["b2"]), prj_g["W"], row(prj_g["b"]),
        gin["W1a"], row(gin["b1a"]), gin["W1b"], row(gin["b1b"]),
        gin["W2a"], row(gin["b2a"]), gin["W2b"], row(gin["b2b"]))


# speedup vs baseline: 1.1956x; 1.1956x over previous
import functools

import jax
import jax.numpy as jnp
from jax import lax
from jax.experimental import pallas as pl
from jax.experimental.pallas import tpu as pltpu
from jax.experimental.pallas import tpu_sc as plsc

_N = 256
_E = 4096
_NC = 2
_NS = 16
_EPT = _E // _NS
_LANES = 16


def _pre_body(x_e, w1_e, av1_e, ea_e, we1_e, ae1_e, we2_e, ae2_e,
              x_g, w1_g, av1_g, ea_g, we1_g, ae1_g, we2_g, ae2_g,
              h1_o, ssrc_o, sdst_o, esc1_o, esc2_o):
    for b, (x, w1, av, ea, we1, ae1, we2, ae2) in enumerate((
            (x_e, w1_e, av1_e, ea_e, we1_e, ae1_e, we2_e, ae2_e),
            (x_g, w1_g, av1_g, ea_g, we1_g, ae1_g, we2_g, ae2_g))):
        h = jnp.dot(x[...], w1[...], preferred_element_type=jnp.float32)
        h1_o[b] = h
        sd = lax.dot_general(av[...], h, (((0,), (1,)), ((), ())),
                             preferred_element_type=jnp.float32)
        ssrc_o[b] = sd[0]
        sdst_o[b] = sd[1]
        wc1 = jnp.dot(we1[...], ae1[...],
                      preferred_element_type=jnp.float32)
        wc2 = jnp.dot(we2[...], ae2[...],
                      preferred_element_type=jnp.float32)
        wcs = jnp.concatenate([wc1, wc2], axis=1)
        e12 = lax.dot_general(wcs, ea[...], (((0,), (1,)), ((), ())),
                              preferred_element_type=jnp.float32)
        esc1_o[b] = e12[0]
        esc2_o[b] = e12[1]


def _tc_pre(*args):
    return pl.pallas_call(
        _pre_body,
        out_shape=[
            jax.ShapeDtypeStruct((2, _N, 512), jnp.float32),
            jax.ShapeDtypeStruct((2, _N), jnp.float32),
            jax.ShapeDtypeStruct((2, _N), jnp.float32),
            jax.ShapeDtypeStruct((2, _E), jnp.float32),
            jax.ShapeDtypeStruct((2, _E), jnp.float32),
        ],
    )(*args)


def _sc_edge(ssrc, sdst, esc, ei, zeros):
    mesh = plsc.VectorSubcoreMesh(
        core_axis_name="c", subcore_axis_name="s",
        num_cores=_NC, num_subcores=_NS)

    @functools.partial(
        pl.kernel,
        out_type=jax.ShapeDtypeStruct((2, _N * _N), jnp.float32),
        mesh=mesh,
        scratch_types=[
            pltpu.VMEM((_N,), jnp.float32),
            pltpu.VMEM((_N,), jnp.float32),
            pltpu.VMEM((_EPT,), jnp.float32),
            pltpu.VMEM((_EPT,), jnp.int32),
            pltpu.VMEM((_EPT,), jnp.int32),
            pltpu.VMEM((2, 128), jnp.float32),
            pltpu.VMEM((2, 128), jnp.int32),
            pltpu.VMEM_SHARED((_N * _N,), jnp.float32),
        ],
        compiler_params=pltpu.CompilerParams(needs_layout_passes=False),
    )
    def k(ssrc_h, sdst_h, esc_h, ei_h, zeros_h, a_out,
          ssrc_v, sdst_v, esc_v, src_v, dst_v, ex_v, idx_v, a_sh):
        c = lax.axis_index("c")
        s = lax.axis_index("s")
        base = s * _EPT
        pltpu.sync_copy(ssrc_h.at[c], ssrc_v)
        pltpu.sync_copy(sdst_h.at[c], sdst_v)
        pltpu.sync_copy(esc_h.at[c, pl.ds(base, _EPT)], esc_v)
        pltpu.sync_copy(ei_h.at[c, 0, pl.ds(base, _EPT)], src_v)
        pltpu.sync_copy(ei_h.at[c, 1, pl.ds(base, _EPT)], dst_v)

        @pl.when(s == 0)
        def _():
            pltpu.sync_copy(zeros_h, a_sh)

        for j in range(_EPT // _LANES):
            si = src_v[pl.ds(j * _LANES, _LANES)]
            di = dst_v[pl.ds(j * _LANES, _LANES)]
            sa = plsc.load_gather(ssrc_v, [si])
            sb = plsc.load_gather(sdst_v, [di])
            t = sa + sb + esc_v[pl.ds(j * _LANES, _LANES)]
            t = jnp.maximum(t, t * jnp.float32(0.2))
            ex_v[j // 8, pl.ds((j % 8) * _LANES, _LANES)] = jnp.exp(t)
            idx_v[j // 8, pl.ds((j % 8) * _LANES, _LANES)] = di * _N + si

        plsc.subcore_barrier()
        for g in range(2):
            pltpu.sync_copy(ex_v.at[g], a_sh.at[idx_v.at[g]], add=True)
        plsc.subcore_barrier()

        @pl.when(s == 0)
        def _():
            pltpu.sync_copy(a_sh, a_out.at[c])

    return k(ssrc, sdst, esc, ei, zeros)


def _mid_body(a, h1, b1_e, b1_g, w2_e, w2_g, av2_e, av2_g,
              h2_o, ssrc_o, sdst_o):
    for b, (b1, w2, av) in enumerate(((b1_e, w2_e, av2_e),
                                      (b1_g, w2_g, av2_g))):
        am = a[b]
        den = jnp.sum(am, axis=1, keepdims=True) + jnp.float32(1e-16)
        m = jnp.dot(am, h1[b], preferred_element_type=jnp.float32) / den
        t = jnp.maximum(m + b1[...], 0.0)
        h2 = jnp.dot(t, w2[...], preferred_element_type=jnp.float32)
        h2_o[b] = h2
        sd = lax.dot_general(av[...], h2, (((0,), (1,)), ((), ())),
                             preferred_element_type=jnp.float32)
        ssrc_o[b] = sd[0]
        sdst_o[b] = sd[1]


def _tc_mid(*args):
    return pl.pallas_call(
        _mid_body,
        out_shape=[
            jax.ShapeDtypeStruct((2, _N, 128), jnp.float32),
            jax.ShapeDtypeStruct((2, _N), jnp.float32),
            jax.ShapeDtypeStruct((2, _N), jnp.float32),
        ],
    )(*args)


def _fin_body(a, h2, b2_e, wp_e, bp_e, b2_g, wp_g, bp_g,
              w1a, b1a, w1b, b1b, w2a, b2a, w2b, b2b, out_o):
    feats = []
    for b, (b2, wp, bp) in enumerate(((b2_e, wp_e, bp_e),
                                      (b2_g, wp_g, bp_g))):
        am = a[b]
        den = jnp.sum(am, axis=1, keepdims=True) + jnp.float32(1e-16)
        o = jnp.dot(am, h2[b], preferred_element_type=jnp.float32) / den
        o = o + b2[...]
        feats.append(jnp.dot(o, wp[...], preferred_element_type=jnp.float32)
                     + bp[...])
    z = jnp.concatenate(feats, axis=0)
    t = z + jnp.sum(z, axis=0, keepdims=True)
    t = jnp.maximum(jnp.dot(t, w1a[...], preferred_element_type=jnp.float32)
                    + b1a[...], 0.0)
    t = jnp.dot(t, w1b[...], preferred_element_type=jnp.float32) + b1b[...]
    t = jnp.maximum(t, 0.0)
    t = t + jnp.sum(t, axis=0, keepdims=True)
    t = jnp.maximum(jnp.dot(t, w2a[...], preferred_element_type=jnp.float32)
                    + b2a[...], 0.0)
    out_o[...] = (jnp.dot(t, w2b[...], preferred_element_type=jnp.float32)
                  + b2b[...])


def _tc_fin(*args):
    return pl.pallas_call(
        _fin_body,
        out_shape=jax.ShapeDtypeStruct((2 * _N, 128), jnp.float32),
    )(*args)


def kernel(emg_x, emg_edge_index, emg_edge_attr,
           eeg_x, eeg_edge_index, eeg_edge_attr, params):
    pe = params["emg_gat"]
    pg = params["eeg_gat"]
    gin = params["gin"]
    row = lambda v: v[None, :]
    col = lambda v: v[:, None]
    sdcols = lambda p, l: jnp.stack([p[f"as{l}"], p[f"ad{l}"]], axis=1)

    h1, ssrc1, sdst1, esc1, esc2 = _tc_pre(
        emg_x, pe["W1"], sdcols(pe, 1), emg_edge_attr,
        pe["We1"], col(pe["ae1"]), pe["We2"], col(pe["ae2"]),
        eeg_x, pg["W1"], sdcols(pg, 1), eeg_edge_attr,
        pg["We1"], col(pg["ae1"]), pg["We2"], col(pg["ae2"]))

    zeros = jnp.zeros((_N * _N,), jnp.float32)
    ei = jnp.stack([emg_edge_index, eeg_edge_index])
    a1 = _sc_edge(ssrc1, sdst1, esc1, ei, zeros)

    h2, ssrc2, sdst2 = _tc_mid(
        a1.reshape(2, _N, _N), h1, row(pe["b1"]), row(pg["b1"]),
        pe["W2"], pg["W2"], sdcols(pe, 2), sdcols(pg, 2))

    a2 = _sc_edge(ssrc2, sdst2, esc2, ei, zeros)

    prj_e = params["emg_proj"]
    prj_g = params["eeg_proj"]
    return _tc_fin(
        a2.reshape(2, _N, _N), h2,
        row(pe["b2"]), prj_e["W"], row(prj_e["b"]),
        row(pg["b2"]), prj_g["W"], row(prj_g["b"]),
        gin["W1a"], row(gin["b1a"]), gin["W1b"], row(gin["b1b"]),
        gin["W2a"], row(gin["b2a"]), gin["W2b"], row(gin["b2b"]))


# speedup vs baseline: 1.3207x; 1.1047x over previous
import functools

import jax
import jax.numpy as jnp
from jax import lax
from jax.experimental import pallas as pl
from jax.experimental.pallas import tpu as pltpu
from jax.experimental.pallas import tpu_sc as plsc

_N = 256
_E = 4096
_NC = 2
_NS = 16
_EPT = _E // _NS
_LANES = 16

_F32 = jnp.float32


def _t_dot(a, b):
    return lax.dot_general(a, b, (((1,), (1,)), ((), ())),
                           preferred_element_type=_F32)


def _pre_body(x_e, w1_e, as1_e, ad1_e, ea_e, we1_e, ae1_e, we2_e, ae2_e,
              x_g, w1_g, as1_g, ad1_g, ea_g, we1_g, ae1_g, we2_g, ae2_g,
              h1_o, ssrc_o, sdst_o, esc1_o, esc2_o):
    for b, (x, w1, a_s, a_d, ea, we1, ae1, we2, ae2) in enumerate((
            (x_e, w1_e, as1_e, ad1_e, ea_e, we1_e, ae1_e, we2_e, ae2_e),
            (x_g, w1_g, as1_g, ad1_g, ea_g, we1_g, ae1_g, we2_g, ae2_g))):
        h = jnp.dot(x[...], w1[...], preferred_element_type=_F32)
        h1_o[b] = h
        av = jnp.concatenate([a_s[...].reshape(1, -1),
                              a_d[...].reshape(1, -1)], axis=0)
        sd = _t_dot(av, h)
        ssrc_o[pl.ds(b * _N, _N)] = sd[0]
        sdst_o[pl.ds(b * _N, _N)] = sd[1]
        wc1 = _t_dot(ae1[...].reshape(1, -1), we1[...])
        wc2 = _t_dot(ae2[...].reshape(1, -1), we2[...])
        e12 = _t_dot(jnp.concatenate([wc1, wc2], axis=0), ea[...])
        esc1_o[pl.ds(b * _E, _E)] = e12[0]
        esc2_o[pl.ds(b * _E, _E)] = e12[1]


def _tc_pre(*args):
    return pl.pallas_call(
        _pre_body,
        out_shape=[
            jax.ShapeDtypeStruct((2, _N, 512), _F32),
            jax.ShapeDtypeStruct((2 * _N,), _F32),
            jax.ShapeDtypeStruct((2 * _N,), _F32),
            jax.ShapeDtypeStruct((2 * _E,), _F32),
            jax.ShapeDtypeStruct((2 * _E,), _F32),
        ],
    )(*args)


def _sc_edge(ssrc, sdst, esc, ei, zeros):
    mesh = plsc.VectorSubcoreMesh(
        core_axis_name="c", subcore_axis_name="s",
        num_cores=_NC, num_subcores=_NS)

    @functools.partial(
        pl.kernel,
        out_type=jax.ShapeDtypeStruct((2 * _N * _N,), _F32),
        mesh=mesh,
        scratch_types=[
            pltpu.VMEM((_N,), _F32),
            pltpu.VMEM((_N,), _F32),
            pltpu.VMEM((_EPT,), _F32),
            pltpu.VMEM((_EPT,), jnp.int32),
            pltpu.VMEM((_EPT,), jnp.int32),
            pltpu.VMEM((2, 128), _F32),
            pltpu.VMEM((2, 128), jnp.int32),
            pltpu.VMEM_SHARED((_N * _N,), _F32),
        ],
        compiler_params=pltpu.CompilerParams(needs_layout_passes=False),
    )
    def k(ssrc_h, sdst_h, esc_h, ei_h, zeros_h, a_out,
          ssrc_v, sdst_v, esc_v, src_v, dst_v, ex_v, idx_v, a_sh):
        c = lax.axis_index("c")
        s = lax.axis_index("s")
        base = s * _EPT
        pltpu.sync_copy(ssrc_h.at[pl.ds(c * _N, _N)], ssrc_v)
        pltpu.sync_copy(sdst_h.at[pl.ds(c * _N, _N)], sdst_v)
        pltpu.sync_copy(esc_h.at[pl.ds(c * _E + base, _EPT)], esc_v)
        pltpu.sync_copy(ei_h.at[pl.ds(c * 2 * _E + base, _EPT)], src_v)
        pltpu.sync_copy(ei_h.at[pl.ds(c * 2 * _E + _E + base, _EPT)], dst_v)

        @pl.when(s == 0)
        def _():
            pltpu.sync_copy(zeros_h, a_sh)

        for j in range(_EPT // _LANES):
            si = src_v[pl.ds(j * _LANES, _LANES)]
            di = dst_v[pl.ds(j * _LANES, _LANES)]
            sa = plsc.load_gather(ssrc_v, [si])
            sb = plsc.load_gather(sdst_v, [di])
            t = sa + sb + esc_v[pl.ds(j * _LANES, _LANES)]
            t = jnp.maximum(t, t * _F32(0.2))
            ex_v[j // 8, pl.ds((j % 8) * _LANES, _LANES)] = jnp.exp(t)
            idx_v[j // 8, pl.ds((j % 8) * _LANES, _LANES)] = di * _N + si

        plsc.subcore_barrier()
        for g in range(2):
            pltpu.sync_copy(ex_v.at[g], a_sh.at[idx_v.at[g]], add=True)
        plsc.subcore_barrier()

        @pl.when(s == 0)
        def _():
            pltpu.sync_copy(a_sh, a_out.at[pl.ds(c * _N * _N, _N * _N)])

    return k(ssrc, sdst, esc, ei, zeros)


def _mid_body(a, h1, b1_e, b1_g, w2_e, w2_g, as2_e, ad2_e, as2_g, ad2_g,
              h2_o, ssrc_o, sdst_o):
    for b, (b1, w2, a_s, a_d) in enumerate(((b1_e, w2_e, as2_e, ad2_e),
                                            (b1_g, w2_g, as2_g, ad2_g))):
        am = a[b]
        den = jnp.sum(am, axis=1, keepdims=True) + _F32(1e-16)
        m = jnp.dot(am, h1[b], preferred_element_type=_F32) / den
        t = jnp.maximum(m + b1[...].reshape(1, -1), 0.0)
        h2 = jnp.dot(t, w2[...], preferred_element_type=_F32)
        h2_o[b] = h2
        av = jnp.concatenate([a_s[...].reshape(1, -1),
                              a_d[...].reshape(1, -1)], axis=0)
        sd = _t_dot(av, h2)
        ssrc_o[pl.ds(b * _N, _N)] = sd[0]
        sdst_o[pl.ds(b * _N, _N)] = sd[1]


def _tc_mid(*args):
    return pl.pallas_call(
        _mid_body,
        out_shape=[
            jax.ShapeDtypeStruct((2, _N, 128), _F32),
            jax.ShapeDtypeStruct((2 * _N,), _F32),
            jax.ShapeDtypeStruct((2 * _N,), _F32),
        ],
    )(*args)


def _fin_body(a, h2, b2_e, wp_e, bp_e, b2_g, wp_g, bp_g,
              w1a, b1a, w1b, b1b, w2a, b2a, w2b, b2b, out_o):
    feats = []
    for b, (b2, wp, bp) in enumerate(((b2_e, wp_e, bp_e),
                                      (b2_g, wp_g, bp_g))):
        am = a[b]
        den = jnp.sum(am, axis=1, keepdims=True) + _F32(1e-16)
        o = jnp.dot(am, h2[b], preferred_element_type=_F32) / den
        o = o + b2[...].reshape(1, -1)
        feats.append(jnp.dot(o, wp[...], preferred_element_type=_F32)
                     + bp[...].reshape(1, -1))
    z = jnp.concatenate(feats, axis=0)
    t = z + jnp.sum(z, axis=0, keepdims=True)
    t = jnp.maximum(jnp.dot(t, w1a[...], preferred_element_type=_F32)
                    + b1a[...].reshape(1, -1), 0.0)
    t = jnp.dot(t, w1b[...], preferred_element_type=_F32) \
        + b1b[...].reshape(1, -1)
    t = jnp.maximum(t, 0.0)
    t = t + jnp.sum(t, axis=0, keepdims=True)
    t = jnp.maximum(jnp.dot(t, w2a[...], preferred_element_type=_F32)
                    + b2a[...].reshape(1, -1), 0.0)
    out_o[...] = (jnp.dot(t, w2b[...], preferred_element_type=_F32)
                  + b2b[...].reshape(1, -1))


def _tc_fin(*args):
    return pl.pallas_call(
        _fin_body,
        out_shape=jax.ShapeDtypeStruct((2 * _N, 128), _F32),
    )(*args)


def kernel(emg_x, emg_edge_index, emg_edge_attr,
           eeg_x, eeg_edge_index, eeg_edge_attr, params):
    pe = params["emg_gat"]
    pg = params["eeg_gat"]
    gin = params["gin"]

    h1, ssrc1, sdst1, esc1, esc2 = _tc_pre(
        emg_x, pe["W1"], pe["as1"], pe["ad1"], emg_edge_attr,
        pe["We1"], pe["ae1"], pe["We2"], pe["ae2"],
        eeg_x, pg["W1"], pg["as1"], pg["ad1"], eeg_edge_attr,
        pg["We1"], pg["ae1"], pg["We2"], pg["ae2"])

    zeros = jnp.zeros((_N * _N,), _F32)
    ei = jnp.concatenate([emg_edge_index.reshape(-1),
                          eeg_edge_index.reshape(-1)])
    a1 = _sc_edge(ssrc1, sdst1, esc1, ei, zeros)

    h2, ssrc2, sdst2 = _tc_mid(
        a1.reshape(2, _N, _N), h1, pe["b1"], pg["b1"],
        pe["W2"], pg["W2"], pe["as2"], pe["ad2"], pg["as2"], pg["ad2"])

    a2 = _sc_edge(ssrc2, sdst2, esc2, ei, zeros)

    prj_e = params["emg_proj"]
    prj_g = params["eeg_proj"]
    return _tc_fin(
        a2.reshape(2, _N, _N), h2,
        pe["b2"], prj_e["W"], prj_e["b"],
        pg["b2"], prj_g["W"], prj_g["b"],
        gin["W1a"], gin["b1a"], gin["W1b"], gin["b1b"],
        gin["W2a"], gin["b2a"], gin["W2b"], gin["b2b"])


# speedup vs baseline: 1.4719x; 1.1145x over previous
import functools

import jax
import jax.numpy as jnp
from jax import lax
from jax.experimental import pallas as pl
from jax.experimental.pallas import tpu as pltpu
from jax.experimental.pallas import tpu_sc as plsc

_N = 256
_E = 4096
_NC = 2
_NS = 16
_EPT = _E // _NS
_LANES = 16

_F32 = jnp.float32


def _t_dot(a, b):
    return lax.dot_general(a, b, (((1,), (1,)), ((), ())),
                           preferred_element_type=_F32)


def _pre_body(x_e, w1_e, as1_e, ad1_e, ea_e, we1_e, ae1_e, we2_e, ae2_e,
              x_g, w1_g, as1_g, ad1_g, ea_g, we1_g, ae1_g, we2_g, ae2_g,
              h1_o, ssrc_o, sdst_o, esc1_o, esc2_o):
    for b, (x, w1, a_s, a_d, ea, we1, ae1, we2, ae2) in enumerate((
            (x_e, w1_e, as1_e, ad1_e, ea_e, we1_e, ae1_e, we2_e, ae2_e),
            (x_g, w1_g, as1_g, ad1_g, ea_g, we1_g, ae1_g, we2_g, ae2_g))):
        h = jnp.dot(x[...], w1[...], preferred_element_type=_F32)
        h1_o[b] = h
        av = jnp.concatenate([a_s[...].reshape(1, -1),
                              a_d[...].reshape(1, -1)], axis=0)
        sd = _t_dot(av, h)
        ssrc_o[pl.ds(b * _N, _N)] = sd[0]
        sdst_o[pl.ds(b * _N, _N)] = sd[1]
        wc1 = _t_dot(ae1[...].reshape(1, -1), we1[...])
        wc2 = _t_dot(ae2[...].reshape(1, -1), we2[...])
        e12 = _t_dot(jnp.concatenate([wc1, wc2], axis=0), ea[...])
        esc1_o[pl.ds(b * _E, _E)] = e12[0]
        esc2_o[pl.ds(b * _E, _E)] = e12[1]


def _tc_pre(*args):
    return pl.pallas_call(
        _pre_body,
        out_shape=[
            jax.ShapeDtypeStruct((2, _N, 512), _F32),
            jax.ShapeDtypeStruct((2 * _N,), _F32),
            jax.ShapeDtypeStruct((2 * _N,), _F32),
            jax.ShapeDtypeStruct((2 * _E,), _F32),
            jax.ShapeDtypeStruct((2 * _E,), _F32),
        ],
    )(*args)


def _sc_edge(ssrc, sdst, esc, ei, zeros):
    mesh = plsc.VectorSubcoreMesh(
        core_axis_name="c", subcore_axis_name="s",
        num_cores=_NC, num_subcores=_NS)

    @functools.partial(
        pl.kernel,
        out_type=jax.ShapeDtypeStruct((2 * _N * _N,), _F32),
        mesh=mesh,
        scratch_types=[
            pltpu.VMEM((_N,), _F32),
            pltpu.VMEM((_N,), _F32),
            pltpu.VMEM((_EPT,), _F32),
            pltpu.VMEM((_EPT,), jnp.int32),
            pltpu.VMEM((_EPT,), jnp.int32),
            pltpu.VMEM((2, 128), _F32),
            pltpu.VMEM((2, 128), jnp.int32),
            pltpu.VMEM_SHARED((_N * _N,), _F32),
            pltpu.SemaphoreType.DMA,
            pltpu.SemaphoreType.DMA,
        ],
        compiler_params=pltpu.CompilerParams(needs_layout_passes=False),
    )
    def k(ssrc_h, sdst_h, esc_h, ei_h, zeros_h, a_out,
          ssrc_v, sdst_v, esc_v, src_v, dst_v, ex_v, idx_v, a_sh,
          sem_in, sem_sc):
        c = lax.axis_index("c")
        s = lax.axis_index("s")
        base = s * _EPT
        cps = [
            pltpu.async_copy(ei_h.at[pl.ds(c * 2 * _E + base, _EPT)],
                             src_v, sem_in),
            pltpu.async_copy(ei_h.at[pl.ds(c * 2 * _E + _E + base, _EPT)],
                             dst_v, sem_in),
            pltpu.async_copy(ssrc_h.at[pl.ds(c * _N, _N)], ssrc_v, sem_in),
            pltpu.async_copy(sdst_h.at[pl.ds(c * _N, _N)], sdst_v, sem_in),
            pltpu.async_copy(esc_h.at[pl.ds(c * _E + base, _EPT)],
                             esc_v, sem_in),
        ]

        @pl.when(s == 0)
        def _():
            pltpu.sync_copy(zeros_h, a_sh)

        for cp in cps:
            cp.wait()

        for j in range(_EPT // _LANES):
            si = src_v[pl.ds(j * _LANES, _LANES)]
            di = dst_v[pl.ds(j * _LANES, _LANES)]
            sa = plsc.load_gather(ssrc_v, [si])
            sb = plsc.load_gather(sdst_v, [di])
            t = sa + sb + esc_v[pl.ds(j * _LANES, _LANES)]
            t = jnp.maximum(t, t * _F32(0.2))
            ex_v[j // 8, pl.ds((j % 8) * _LANES, _LANES)] = jnp.exp(t)
            idx_v[j // 8, pl.ds((j % 8) * _LANES, _LANES)] = di * _N + si

        plsc.subcore_barrier()
        d0 = pltpu.async_copy(ex_v.at[0], a_sh.at[idx_v.at[0]], sem_sc,
                              add=True)
        d1 = pltpu.async_copy(ex_v.at[1], a_sh.at[idx_v.at[1]], sem_sc,
                              add=True)
        d0.wait()
        d1.wait()
        plsc.subcore_barrier()

        @pl.when(s == 0)
        def _():
            pltpu.sync_copy(a_sh, a_out.at[pl.ds(c * _N * _N, _N * _N)])

    return k(ssrc, sdst, esc, ei, zeros)


def _mid_body(a, h1, b1_e, b1_g, w2_e, w2_g, as2_e, ad2_e, as2_g, ad2_g,
              h2_o, ssrc_o, sdst_o):
    for b, (b1, w2, a_s, a_d) in enumerate(((b1_e, w2_e, as2_e, ad2_e),
                                            (b1_g, w2_g, as2_g, ad2_g))):
        am = a[b]
        den = jnp.sum(am, axis=1, keepdims=True) + _F32(1e-16)
        m = jnp.dot(am, h1[b], preferred_element_type=_F32) / den
        t = jnp.maximum(m + b1[...].reshape(1, -1), 0.0)
        h2 = jnp.dot(t, w2[...], preferred_element_type=_F32)
        h2_o[b] = h2
        av = jnp.concatenate([a_s[...].reshape(1, -1),
                              a_d[...].reshape(1, -1)], axis=0)
        sd = _t_dot(av, h2)
        ssrc_o[pl.ds(b * _N, _N)] = sd[0]
        sdst_o[pl.ds(b * _N, _N)] = sd[1]


def _tc_mid(*args):
    return pl.pallas_call(
        _mid_body,
        out_shape=[
            jax.ShapeDtypeStruct((2, _N, 128), _F32),
            jax.ShapeDtypeStruct((2 * _N,), _F32),
            jax.ShapeDtypeStruct((2 * _N,), _F32),
        ],
    )(*args)


def _fin_body(a, h2, b2_e, wp_e, bp_e, b2_g, wp_g, bp_g,
              w1a, b1a, w1b, b1b, w2a, b2a, w2b, b2b, out_o):
    feats = []
    for b, (b2, wp, bp) in enumerate(((b2_e, wp_e, bp_e),
                                      (b2_g, wp_g, bp_g))):
        am = a[b]
        den = jnp.sum(am, axis=1, keepdims=True) + _F32(1e-16)
        o = jnp.dot(am, h2[b], preferred_element_type=_F32) / den
        o = o + b2[...].reshape(1, -1)
        feats.append(jnp.dot(o, wp[...], preferred_element_type=_F32)
                     + bp[...].reshape(1, -1))
    z = jnp.concatenate(feats, axis=0)
    t = z + jnp.sum(z, axis=0, keepdims=True)
    t = jnp.maximum(jnp.dot(t, w1a[...], preferred_element_type=_F32)
                    + b1a[...].reshape(1, -1), 0.0)
    t = jnp.dot(t, w1b[...], preferred_element_type=_F32) \
        + b1b[...].reshape(1, -1)
    t = jnp.maximum(t, 0.0)
    t = t + jnp.sum(t, axis=0, keepdims=True)
    t = jnp.maximum(jnp.dot(t, w2a[...], preferred_element_type=_F32)
                    + b2a[...].reshape(1, -1), 0.0)
    out_o[...] = (jnp.dot(t, w2b[...], preferred_element_type=_F32)
                  + b2b[...].reshape(1, -1))


def _tc_fin(*args):
    return pl.pallas_call(
        _fin_body,
        out_shape=jax.ShapeDtypeStruct((2 * _N, 128), _F32),
    )(*args)


def kernel(emg_x, emg_edge_index, emg_edge_attr,
           eeg_x, eeg_edge_index, eeg_edge_attr, params):
    pe = params["emg_gat"]
    pg = params["eeg_gat"]
    gin = params["gin"]

    h1, ssrc1, sdst1, esc1, esc2 = _tc_pre(
        emg_x, pe["W1"], pe["as1"], pe["ad1"], emg_edge_attr,
        pe["We1"], pe["ae1"], pe["We2"], pe["ae2"],
        eeg_x, pg["W1"], pg["as1"], pg["ad1"], eeg_edge_attr,
        pg["We1"], pg["ae1"], pg["We2"], pg["ae2"])

    zeros = jnp.zeros((_N * _N,), _F32)
    ei = jnp.concatenate([emg_edge_index.reshape(-1),
                          eeg_edge_index.reshape(-1)])
    a1 = _sc_edge(ssrc1, sdst1, esc1, ei, zeros)

    h2, ssrc2, sdst2 = _tc_mid(
        a1.reshape(2, _N, _N), h1, pe["b1"], pg["b1"],
        pe["W2"], pg["W2"], pe["as2"], pe["ad2"], pg["as2"], pg["ad2"])

    a2 = _sc_edge(ssrc2, sdst2, esc2, ei, zeros)

    prj_e = params["emg_proj"]
    prj_g = params["eeg_proj"]
    return _tc_fin(
        a2.reshape(2, _N, _N), h2,
        pe["b2"], prj_e["W"], prj_e["b"],
        pg["b2"], prj_g["W"], prj_g["b"],
        gin["W1a"], gin["b1a"], gin["W1b"], gin["b1b"],
        gin["W2a"], gin["b2a"], gin["W2b"], gin["b2b"])


# speedup vs baseline: 1.5900x; 1.0802x over previous
import functools

import jax
import jax.numpy as jnp
from jax import lax
from jax.experimental import pallas as pl
from jax.experimental.pallas import tpu as pltpu
from jax.experimental.pallas import tpu_sc as plsc

_N = 256
_E = 4096
_NC = 2
_NS = 16
_EPT = _E // _NS
_LANES = 16

_F32 = jnp.float32


def _t_dot(a, b):
    return lax.dot_general(a, b, (((1,), (1,)), ((), ())),
                           preferred_element_type=_F32)


_KB = 4
_KC = 2048 // _KB


def _pre_body(x_e, w1_e, as1_e, ad1_e, eat_e, we1_e, ae1_e, we2_e, ae2_e,
              x_g, w1_g, as1_g, ad1_g, eat_g, we1_g, ae1_g, we2_g, ae2_g,
              h1_o, ssrc_o, sdst_o, esc1_o, esc2_o):
    k = pl.program_id(0)
    branches = ((x_e, w1_e, as1_e, ad1_e, eat_e, we1_e, ae1_e, we2_e, ae2_e),
                (x_g, w1_g, as1_g, ad1_g, eat_g, we1_g, ae1_g, we2_g, ae2_g))
    for b, (x, w1, a_s, a_d, eat, we1, ae1, we2, ae2) in enumerate(branches):
        part = jnp.dot(x[...], w1[...], preferred_element_type=_F32)

        @pl.when(k == 0)
        def _(part=part, b=b):
            h1_o[b] = part

        @pl.when(k > 0)
        def _(part=part, b=b):
            h1_o[b] = h1_o[b] + part

    @pl.when(k == 0)
    def _():
        for b, (x, w1, a_s, a_d, eat, we1, ae1, we2, ae2) in \
                enumerate(branches):
            wc1 = _t_dot(ae1[...].reshape(1, -1), we1[...])
            wc2 = _t_dot(ae2[...].reshape(1, -1), we2[...])
            e12 = jnp.dot(jnp.concatenate([wc1, wc2], axis=0), eat[...],
                          preferred_element_type=_F32)
            esc1_o[pl.ds(b * _E, _E)] = e12[0]
            esc2_o[pl.ds(b * _E, _E)] = e12[1]

    @pl.when(k == _KB - 1)
    def _():
        for b, (x, w1, a_s, a_d, eat, we1, ae1, we2, ae2) in \
                enumerate(branches):
            av = jnp.concatenate([a_s[...].reshape(1, -1),
                                  a_d[...].reshape(1, -1)], axis=0)
            sd = _t_dot(av, h1_o[b])
            ssrc_o[pl.ds(b * _N, _N)] = sd[0]
            sdst_o[pl.ds(b * _N, _N)] = sd[1]


def _tc_pre(*args):
    full = lambda shape: pl.BlockSpec(shape, lambda k: (0,) * len(shape))
    xs = pl.BlockSpec((_N, _KC), lambda k: (0, k))
    ws = pl.BlockSpec((_KC, 512), lambda k: (k, 0))
    per_branch = [xs, ws, full((512,)), full((512,)), full((16, _E)),
                  full((16, 512)), full((512,)), full((16, 128)),
                  full((128,))]
    return pl.pallas_call(
        _pre_body,
        grid=(_KB,),
        in_specs=per_branch + per_branch,
        out_specs=[
            pl.BlockSpec((2, _N, 512), lambda k: (0, 0, 0)),
            pl.BlockSpec((2 * _N,), lambda k: (0,)),
            pl.BlockSpec((2 * _N,), lambda k: (0,)),
            pl.BlockSpec((2 * _E,), lambda k: (0,)),
            pl.BlockSpec((2 * _E,), lambda k: (0,)),
        ],
        out_shape=[
            jax.ShapeDtypeStruct((2, _N, 512), _F32),
            jax.ShapeDtypeStruct((2 * _N,), _F32),
            jax.ShapeDtypeStruct((2 * _N,), _F32),
            jax.ShapeDtypeStruct((2 * _E,), _F32),
            jax.ShapeDtypeStruct((2 * _E,), _F32),
        ],
        compiler_params=pltpu.CompilerParams(
            dimension_semantics=("arbitrary",)),
    )(*args)


def _sc_edge(ssrc, sdst, esc, ei, zeros):
    mesh = plsc.VectorSubcoreMesh(
        core_axis_name="c", subcore_axis_name="s",
        num_cores=_NC, num_subcores=_NS)

    @functools.partial(
        pl.kernel,
        out_type=jax.ShapeDtypeStruct((2 * _N * _N,), _F32),
        mesh=mesh,
        scratch_types=[
            pltpu.VMEM((_N,), _F32),
            pltpu.VMEM((_N,), _F32),
            pltpu.VMEM((_EPT,), _F32),
            pltpu.VMEM((_EPT,), jnp.int32),
            pltpu.VMEM((_EPT,), jnp.int32),
            pltpu.VMEM((2, 128), _F32),
            pltpu.VMEM((2, 128), jnp.int32),
            pltpu.VMEM_SHARED((_N * _N,), _F32),
            pltpu.SemaphoreType.DMA,
            pltpu.SemaphoreType.DMA,
        ],
        compiler_params=pltpu.CompilerParams(needs_layout_passes=False),
    )
    def k(ssrc_h, sdst_h, esc_h, ei_h, zeros_h, a_out,
          ssrc_v, sdst_v, esc_v, src_v, dst_v, ex_v, idx_v, a_sh,
          sem_in, sem_sc):
        c = lax.axis_index("c")
        s = lax.axis_index("s")
        base = s * _EPT
        cps = [
            pltpu.async_copy(ei_h.at[pl.ds(c * 2 * _E + base, _EPT)],
                             src_v, sem_in),
            pltpu.async_copy(ei_h.at[pl.ds(c * 2 * _E + _E + base, _EPT)],
                             dst_v, sem_in),
            pltpu.async_copy(ssrc_h.at[pl.ds(c * _N, _N)], ssrc_v, sem_in),
            pltpu.async_copy(sdst_h.at[pl.ds(c * _N, _N)], sdst_v, sem_in),
            pltpu.async_copy(esc_h.at[pl.ds(c * _E + base, _EPT)],
                             esc_v, sem_in),
        ]

        @pl.when(s == 0)
        def _():
            pltpu.sync_copy(zeros_h, a_sh)

        for cp in cps:
            cp.wait()

        for j in range(_EPT // _LANES):
            si = src_v[pl.ds(j * _LANES, _LANES)]
            di = dst_v[pl.ds(j * _LANES, _LANES)]
            sa = plsc.load_gather(ssrc_v, [si])
            sb = plsc.load_gather(sdst_v, [di])
            t = sa + sb + esc_v[pl.ds(j * _LANES, _LANES)]
            t = jnp.maximum(t, t * _F32(0.2))
            ex_v[j // 8, pl.ds((j % 8) * _LANES, _LANES)] = jnp.exp(t)
            idx_v[j // 8, pl.ds((j % 8) * _LANES, _LANES)] = di * _N + si

        plsc.subcore_barrier()
        d0 = pltpu.async_copy(ex_v.at[0], a_sh.at[idx_v.at[0]], sem_sc,
                              add=True)
        d1 = pltpu.async_copy(ex_v.at[1], a_sh.at[idx_v.at[1]], sem_sc,
                              add=True)
        d0.wait()
        d1.wait()
        plsc.subcore_barrier()

        @pl.when(s == 0)
        def _():
            pltpu.sync_copy(a_sh, a_out.at[pl.ds(c * _N * _N, _N * _N)])

    return k(ssrc, sdst, esc, ei, zeros)


def _mid_body(a, h1, b1_e, b1_g, w2_e, w2_g, as2_e, ad2_e, as2_g, ad2_g,
              h2_o, ssrc_o, sdst_o):
    for b, (b1, w2, a_s, a_d) in enumerate(((b1_e, w2_e, as2_e, ad2_e),
                                            (b1_g, w2_g, as2_g, ad2_g))):
        am = a[b]
        den = jnp.sum(am, axis=1, keepdims=True) + _F32(1e-16)
        m = jnp.dot(am, h1[b], preferred_element_type=_F32) / den
        t = jnp.maximum(m + b1[...].reshape(1, -1), 0.0)
        h2 = jnp.dot(t, w2[...], preferred_element_type=_F32)
        h2_o[b] = h2
        av = jnp.concatenate([a_s[...].reshape(1, -1),
                              a_d[...].reshape(1, -1)], axis=0)
        sd = _t_dot(av, h2)
        ssrc_o[pl.ds(b * _N, _N)] = sd[0]
        sdst_o[pl.ds(b * _N, _N)] = sd[1]


def _tc_mid(*args):
    return pl.pallas_call(
        _mid_body,
        out_shape=[
            jax.ShapeDtypeStruct((2, _N, 128), _F32),
            jax.ShapeDtypeStruct((2 * _N,), _F32),
            jax.ShapeDtypeStruct((2 * _N,), _F32),
        ],
    )(*args)


def _fin_body(a, h2, b2_e, wp_e, bp_e, b2_g, wp_g, bp_g,
              w1a, b1a, w1b, b1b, w2a, b2a, w2b, b2b, out_o):
    feats = []
    for b, (b2, wp, bp) in enumerate(((b2_e, wp_e, bp_e),
                                      (b2_g, wp_g, bp_g))):
        am = a[b]
        den = jnp.sum(am, axis=1, keepdims=True) + _F32(1e-16)
        o = jnp.dot(am, h2[b], preferred_element_type=_F32) / den
        o = o + b2[...].reshape(1, -1)
        feats.append(jnp.dot(o, wp[...], preferred_element_type=_F32)
                     + bp[...].reshape(1, -1))
    z = jnp.concatenate(feats, axis=0)
    t = z + jnp.sum(z, axis=0, keepdims=True)
    t = jnp.maximum(jnp.dot(t, w1a[...], preferred_element_type=_F32)
                    + b1a[...].reshape(1, -1), 0.0)
    t = jnp.dot(t, w1b[...], preferred_element_type=_F32) \
        + b1b[...].reshape(1, -1)
    t = jnp.maximum(t, 0.0)
    t = t + jnp.sum(t, axis=0, keepdims=True)
    t = jnp.maximum(jnp.dot(t, w2a[...], preferred_element_type=_F32)
                    + b2a[...].reshape(1, -1), 0.0)
    out_o[...] = (jnp.dot(t, w2b[...], preferred_element_type=_F32)
                  + b2b[...].reshape(1, -1))


def _tc_fin(*args):
    return pl.pallas_call(
        _fin_body,
        out_shape=jax.ShapeDtypeStruct((2 * _N, 128), _F32),
    )(*args)


def kernel(emg_x, emg_edge_index, emg_edge_attr,
           eeg_x, eeg_edge_index, eeg_edge_attr, params):
    pe = params["emg_gat"]
    pg = params["eeg_gat"]
    gin = params["gin"]

    h1, ssrc1, sdst1, esc1, esc2 = _tc_pre(
        emg_x, pe["W1"], pe["as1"], pe["ad1"], emg_edge_attr.T,
        pe["We1"], pe["ae1"], pe["We2"], pe["ae2"],
        eeg_x, pg["W1"], pg["as1"], pg["ad1"], eeg_edge_attr.T,
        pg["We1"], pg["ae1"], pg["We2"], pg["ae2"])

    zeros = jnp.zeros((_N * _N,), _F32)
    ei = jnp.concatenate([emg_edge_index.reshape(-1),
                          eeg_edge_index.reshape(-1)])
    a1 = _sc_edge(ssrc1, sdst1, esc1, ei, zeros)

    h2, ssrc2, sdst2 = _tc_mid(
        a1.reshape(2, _N, _N), h1, pe["b1"], pg["b1"],
        pe["W2"], pg["W2"], pe["as2"], pe["ad2"], pg["as2"], pg["ad2"])

    a2 = _sc_edge(ssrc2, sdst2, esc2, ei, zeros)

    prj_e = params["emg_proj"]
    prj_g = params["eeg_proj"]
    return _tc_fin(
        a2.reshape(2, _N, _N), h2,
        pe["b2"], prj_e["W"], prj_e["b"],
        pg["b2"], prj_g["W"], prj_g["b"],
        gin["W1a"], gin["b1a"], gin["W1b"], gin["b1b"],
        gin["W2a"], gin["b2a"], gin["W2b"], gin["b2b"])


# speedup vs baseline: 1.6045x; 1.0091x over previous
import functools

import jax
import jax.numpy as jnp
from jax import lax
from jax.experimental import pallas as pl
from jax.experimental.pallas import tpu as pltpu
from jax.experimental.pallas import tpu_sc as plsc

_N = 256
_E = 4096
_NC = 2
_NS = 16
_EPT = _E // _NS
_LANES = 16

_F32 = jnp.float32


def _t_dot(a, b):
    return lax.dot_general(a, b, (((1,), (1,)), ((), ())),
                           preferred_element_type=_F32)


_KB = 4
_KC = 2048 // _KB


def _pre_body(x_e, w1_e, as1_e, ad1_e, eat_e, we1_e, ae1_e, we2_e, ae2_e,
              x_g, w1_g, as1_g, ad1_g, eat_g, we1_g, ae1_g, we2_g, ae2_g,
              h1_o, ssrc_o, sdst_o, esc1_o, esc2_o):
    k = pl.program_id(0)
    branches = ((x_e, w1_e, as1_e, ad1_e, eat_e, we1_e, ae1_e, we2_e, ae2_e),
                (x_g, w1_g, as1_g, ad1_g, eat_g, we1_g, ae1_g, we2_g, ae2_g))
    for b, (x, w1, a_s, a_d, eat, we1, ae1, we2, ae2) in enumerate(branches):
        part = jnp.dot(x[...], w1[...], preferred_element_type=_F32)

        @pl.when(k == 0)
        def _(part=part, b=b):
            h1_o[b] = part

        @pl.when(k > 0)
        def _(part=part, b=b):
            h1_o[b] = h1_o[b] + part

    @pl.when(k == 0)
    def _():
        for b, (x, w1, a_s, a_d, eat, we1, ae1, we2, ae2) in \
                enumerate(branches):
            wc1 = _t_dot(ae1[...].reshape(1, -1), we1[...])
            wc2 = _t_dot(ae2[...].reshape(1, -1), we2[...])
            e12 = jnp.dot(jnp.concatenate([wc1, wc2], axis=0), eat[...],
                          preferred_element_type=_F32)
            esc1_o[pl.ds(b * _E, _E)] = e12[0]
            esc2_o[pl.ds(b * _E, _E)] = e12[1]

    @pl.when(k == _KB - 1)
    def _():
        for b, (x, w1, a_s, a_d, eat, we1, ae1, we2, ae2) in \
                enumerate(branches):
            av = jnp.concatenate([a_s[...].reshape(1, -1),
                                  a_d[...].reshape(1, -1)], axis=0)
            sd = _t_dot(av, h1_o[b])
            ssrc_o[pl.ds(b * _N, _N)] = sd[0]
            sdst_o[pl.ds(b * _N, _N)] = sd[1]


def _tc_pre(*args):
    full = lambda shape: pl.BlockSpec(shape, lambda k: (0,) * len(shape))
    xs = pl.BlockSpec((_N, _KC), lambda k: (0, k))
    ws = pl.BlockSpec((_KC, 512), lambda k: (k, 0))
    per_branch = [xs, ws, full((512,)), full((512,)), full((16, _E)),
                  full((16, 512)), full((512,)), full((16, 128)),
                  full((128,))]
    return pl.pallas_call(
        _pre_body,
        grid=(_KB,),
        in_specs=per_branch + per_branch,
        out_specs=[
            pl.BlockSpec((2, _N, 512), lambda k: (0, 0, 0)),
            pl.BlockSpec((2 * _N,), lambda k: (0,)),
            pl.BlockSpec((2 * _N,), lambda k: (0,)),
            pl.BlockSpec((2 * _E,), lambda k: (0,)),
            pl.BlockSpec((2 * _E,), lambda k: (0,)),
        ],
        out_shape=[
            jax.ShapeDtypeStruct((2, _N, 512), _F32),
            jax.ShapeDtypeStruct((2 * _N,), _F32),
            jax.ShapeDtypeStruct((2 * _N,), _F32),
            jax.ShapeDtypeStruct((2 * _E,), _F32),
            jax.ShapeDtypeStruct((2 * _E,), _F32),
        ],
        compiler_params=pltpu.CompilerParams(
            dimension_semantics=("arbitrary",)),
    )(*args)


def _sc_edge(ssrc, sdst, esc, ei, zeros):
    mesh = plsc.VectorSubcoreMesh(
        core_axis_name="c", subcore_axis_name="s",
        num_cores=_NC, num_subcores=_NS)

    @functools.partial(
        pl.kernel,
        out_type=jax.ShapeDtypeStruct((2 * _N * _N,), _F32),
        mesh=mesh,
        scratch_types=[
            pltpu.VMEM((_N,), _F32),
            pltpu.VMEM((_N,), _F32),
            pltpu.VMEM((_EPT,), _F32),
            pltpu.VMEM((_EPT,), jnp.int32),
            pltpu.VMEM((_EPT,), jnp.int32),
            pltpu.VMEM((2, 128), _F32),
            pltpu.VMEM((2, 128), jnp.int32),
            pltpu.VMEM_SHARED((_N * _N,), _F32),
            pltpu.SemaphoreType.DMA,
            pltpu.SemaphoreType.DMA,
        ],
        compiler_params=pltpu.CompilerParams(needs_layout_passes=False),
    )
    def k(ssrc_h, sdst_h, esc_h, ei_h, zeros_h, a_out,
          ssrc_v, sdst_v, esc_v, src_v, dst_v, ex_v, idx_v, a_sh,
          sem_in, sem_sc):
        c = lax.axis_index("c")
        s = lax.axis_index("s")
        base = s * _EPT
        cps = [
            pltpu.async_copy(ei_h.at[pl.ds(c * 2 * _E + base, _EPT)],
                             src_v, sem_in),
            pltpu.async_copy(ei_h.at[pl.ds(c * 2 * _E + _E + base, _EPT)],
                             dst_v, sem_in),
            pltpu.async_copy(ssrc_h.at[pl.ds(c * _N, _N)], ssrc_v, sem_in),
            pltpu.async_copy(sdst_h.at[pl.ds(c * _N, _N)], sdst_v, sem_in),
            pltpu.async_copy(esc_h.at[pl.ds(c * _E + base, _EPT)],
                             esc_v, sem_in),
        ]

        @pl.when(s == 0)
        def _():
            pltpu.sync_copy(zeros_h, a_sh)

        for cp in cps:
            cp.wait()

        @pl.loop(0, _EPT // _LANES)
        def _(j):
            si = src_v[pl.ds(j * _LANES, _LANES)]
            di = dst_v[pl.ds(j * _LANES, _LANES)]
            sa = plsc.load_gather(ssrc_v, [si])
            sb = plsc.load_gather(sdst_v, [di])
            t = sa + sb + esc_v[pl.ds(j * _LANES, _LANES)]
            t = jnp.maximum(t, t * _F32(0.2))
            ex_v[j // 8, pl.ds((j % 8) * _LANES, _LANES)] = jnp.exp(t)
            idx_v[j // 8, pl.ds((j % 8) * _LANES, _LANES)] = di * _N + si

        plsc.subcore_barrier()
        d0 = pltpu.async_copy(ex_v.at[0], a_sh.at[idx_v.at[0]], sem_sc,
                              add=True)
        d1 = pltpu.async_copy(ex_v.at[1], a_sh.at[idx_v.at[1]], sem_sc,
                              add=True)
        d0.wait()
        d1.wait()
        plsc.subcore_barrier()

        @pl.when(s == 0)
        def _():
            pltpu.sync_copy(a_sh, a_out.at[pl.ds(c * _N * _N, _N * _N)])

    return k(ssrc, sdst, esc, ei, zeros)


def _mid_body(a, h1, b1_e, b1_g, w2_e, w2_g, as2_e, ad2_e, as2_g, ad2_g,
              h2_o, ssrc_o, sdst_o):
    for b, (b1, w2, a_s, a_d) in enumerate(((b1_e, w2_e, as2_e, ad2_e),
                                            (b1_g, w2_g, as2_g, ad2_g))):
        am = a[b]
        den = jnp.sum(am, axis=1, keepdims=True) + _F32(1e-16)
        m = jnp.dot(am, h1[b], preferred_element_type=_F32) / den
        t = jnp.maximum(m + b1[...].reshape(1, -1), 0.0)
        h2 = jnp.dot(t, w2[...], preferred_element_type=_F32)
        h2_o[b] = h2
        av = jnp.concatenate([a_s[...].reshape(1, -1),
                              a_d[...].reshape(1, -1)], axis=0)
        sd = _t_dot(av, h2)
        ssrc_o[pl.ds(b * _N, _N)] = sd[0]
        sdst_o[pl.ds(b * _N, _N)] = sd[1]


def _tc_mid(*args):
    return pl.pallas_call(
        _mid_body,
        out_shape=[
            jax.ShapeDtypeStruct((2, _N, 128), _F32),
            jax.ShapeDtypeStruct((2 * _N,), _F32),
            jax.ShapeDtypeStruct((2 * _N,), _F32),
        ],
    )(*args)


def _fin_body(a, h2, b2_e, wp_e, bp_e, b2_g, wp_g, bp_g,
              w1a, b1a, w1b, b1b, w2a, b2a, w2b, b2b, out_o):
    feats = []
    for b, (b2, wp, bp) in enumerate(((b2_e, wp_e, bp_e),
                                      (b2_g, wp_g, bp_g))):
        am = a[b]
        den = jnp.sum(am, axis=1, keepdims=True) + _F32(1e-16)
        o = jnp.dot(am, h2[b], preferred_element_type=_F32) / den
        o = o + b2[...].reshape(1, -1)
        feats.append(jnp.dot(o, wp[...], preferred_element_type=_F32)
                     + bp[...].reshape(1, -1))
    z = jnp.concatenate(feats, axis=0)
    t = z + jnp.sum(z, axis=0, keepdims=True)
    t = jnp.maximum(jnp.dot(t, w1a[...], preferred_element_type=_F32)
                    + b1a[...].reshape(1, -1), 0.0)
    t = jnp.dot(t, w1b[...], preferred_element_type=_F32) \
        + b1b[...].reshape(1, -1)
    t = jnp.maximum(t, 0.0)
    t = t + jnp.sum(t, axis=0, keepdims=True)
    t = jnp.maximum(jnp.dot(t, w2a[...], preferred_element_type=_F32)
                    + b2a[...].reshape(1, -1), 0.0)
    out_o[...] = (jnp.dot(t, w2b[...], preferred_element_type=_F32)
                  + b2b[...].reshape(1, -1))


def _tc_fin(*args):
    return pl.pallas_call(
        _fin_body,
        out_shape=jax.ShapeDtypeStruct((2 * _N, 128), _F32),
    )(*args)


def kernel(emg_x, emg_edge_index, emg_edge_attr,
           eeg_x, eeg_edge_index, eeg_edge_attr, params):
    pe = params["emg_gat"]
    pg = params["eeg_gat"]
    gin = params["gin"]

    h1, ssrc1, sdst1, esc1, esc2 = _tc_pre(
        emg_x, pe["W1"], pe["as1"], pe["ad1"], emg_edge_attr.T,
        pe["We1"], pe["ae1"], pe["We2"], pe["ae2"],
        eeg_x, pg["W1"], pg["as1"], pg["ad1"], eeg_edge_attr.T,
        pg["We1"], pg["ae1"], pg["We2"], pg["ae2"])

    zeros = jnp.zeros((_N * _N,), _F32)
    ei = jnp.concatenate([emg_edge_index.reshape(-1),
                          eeg_edge_index.reshape(-1)])
    a1 = _sc_edge(ssrc1, sdst1, esc1, ei, zeros)

    h2, ssrc2, sdst2 = _tc_mid(
        a1.reshape(2, _N, _N), h1, pe["b1"], pg["b1"],
        pe["W2"], pg["W2"], pe["as2"], pe["ad2"], pg["as2"], pg["ad2"])

    a2 = _sc_edge(ssrc2, sdst2, esc2, ei, zeros)

    prj_e = params["emg_proj"]
    prj_g = params["eeg_proj"]
    return _tc_fin(
        a2.reshape(2, _N, _N), h2,
        pe["b2"], prj_e["W"], prj_e["b"],
        pg["b2"], prj_g["W"], prj_g["b"],
        gin["W1a"], gin["b1a"], gin["W1b"], gin["b1b"],
        gin["W2a"], gin["b2a"], gin["W2b"], gin["b2b"])


# speedup vs baseline: 1.6061x; 1.0010x over previous
import functools

import jax
import jax.numpy as jnp
from jax import lax
from jax.experimental import pallas as pl
from jax.experimental.pallas import tpu as pltpu
from jax.experimental.pallas import tpu_sc as plsc

_N = 256
_E = 4096
_NC = 2
_NS = 16
_EPT = _E // _NS
_LANES = 16

_F32 = jnp.float32


def _t_dot(a, b):
    return lax.dot_general(a, b, (((1,), (1,)), ((), ())),
                           preferred_element_type=_F32)


_KB = 4
_KC = 2048 // _KB


def _pre_body(x_e, w1_e, as1_e, ad1_e, eat_e, we1_e, ae1_e, we2_e, ae2_e,
              x_g, w1_g, as1_g, ad1_g, eat_g, we1_g, ae1_g, we2_g, ae2_g,
              h1_o, ssrc_o, sdst_o, esc1_o, esc2_o):
    k = pl.program_id(0)
    branches = ((x_e, w1_e, as1_e, ad1_e, eat_e, we1_e, ae1_e, we2_e, ae2_e),
                (x_g, w1_g, as1_g, ad1_g, eat_g, we1_g, ae1_g, we2_g, ae2_g))
    for b, (x, w1, a_s, a_d, eat, we1, ae1, we2, ae2) in enumerate(branches):
        part = jnp.dot(x[...], w1[...], preferred_element_type=_F32)

        @pl.when(k == 0)
        def _(part=part, b=b):
            h1_o[b] = part

        @pl.when(k > 0)
        def _(part=part, b=b):
            h1_o[b] = h1_o[b] + part

    @pl.when(k == 0)
    def _():
        for b, (x, w1, a_s, a_d, eat, we1, ae1, we2, ae2) in \
                enumerate(branches):
            wc1 = _t_dot(ae1[...].reshape(1, -1), we1[...])
            wc2 = _t_dot(ae2[...].reshape(1, -1), we2[...])
            e12 = jnp.dot(jnp.concatenate([wc1, wc2], axis=0), eat[...],
                          preferred_element_type=_F32)
            esc1_o[pl.ds(b * _E, _E)] = e12[0]
            esc2_o[pl.ds(b * _E, _E)] = e12[1]

    @pl.when(k == _KB - 1)
    def _():
        for b, (x, w1, a_s, a_d, eat, we1, ae1, we2, ae2) in \
                enumerate(branches):
            av = jnp.concatenate([a_s[...].reshape(1, -1),
                                  a_d[...].reshape(1, -1)], axis=0)
            sd = _t_dot(av, h1_o[b])
            ssrc_o[pl.ds(b * _N, _N)] = sd[0]
            sdst_o[pl.ds(b * _N, _N)] = sd[1]


def _tc_pre(*args):
    full = lambda shape: pl.BlockSpec(shape, lambda k: (0,) * len(shape))
    xs = pl.BlockSpec((_N, _KC), lambda k: (0, k))
    ws = pl.BlockSpec((_KC, 512), lambda k: (k, 0))
    per_branch = [xs, ws, full((512,)), full((512,)), full((16, _E)),
                  full((16, 512)), full((512,)), full((16, 128)),
                  full((128,))]
    return pl.pallas_call(
        _pre_body,
        grid=(_KB,),
        in_specs=per_branch + per_branch,
        out_specs=[
            pl.BlockSpec((2, _N, 512), lambda k: (0, 0, 0)),
            pl.BlockSpec((2 * _N,), lambda k: (0,)),
            pl.BlockSpec((2 * _N,), lambda k: (0,)),
            pl.BlockSpec((2 * _E,), lambda k: (0,)),
            pl.BlockSpec((2 * _E,), lambda k: (0,)),
        ],
        out_shape=[
            jax.ShapeDtypeStruct((2, _N, 512), _F32),
            jax.ShapeDtypeStruct((2 * _N,), _F32),
            jax.ShapeDtypeStruct((2 * _N,), _F32),
            jax.ShapeDtypeStruct((2 * _E,), _F32),
            jax.ShapeDtypeStruct((2 * _E,), _F32),
        ],
        compiler_params=pltpu.CompilerParams(
            dimension_semantics=("arbitrary",)),
    )(*args)


@functools.cache
def _sc_edge_kernel():
    mesh = plsc.VectorSubcoreMesh(
        core_axis_name="c", subcore_axis_name="s",
        num_cores=_NC, num_subcores=_NS)

    @functools.partial(
        pl.kernel,
        out_type=jax.ShapeDtypeStruct((2 * _N * _N,), _F32),
        mesh=mesh,
        scratch_types=[
            pltpu.VMEM((_N,), _F32),
            pltpu.VMEM((_N,), _F32),
            pltpu.VMEM((_EPT,), _F32),
            pltpu.VMEM((_EPT,), jnp.int32),
            pltpu.VMEM((_EPT,), jnp.int32),
            pltpu.VMEM((2, 128), _F32),
            pltpu.VMEM((2, 128), jnp.int32),
            pltpu.VMEM_SHARED((_N * _N,), _F32),
            pltpu.SemaphoreType.DMA,
            pltpu.SemaphoreType.DMA,
        ],
        compiler_params=pltpu.CompilerParams(needs_layout_passes=False),
    )
    def k(ssrc_h, sdst_h, esc_h, ei_h, zeros_h, a_out,
          ssrc_v, sdst_v, esc_v, src_v, dst_v, ex_v, idx_v, a_sh,
          sem_in, sem_sc):
        c = lax.axis_index("c")
        s = lax.axis_index("s")
        base = s * _EPT
        cps = [
            pltpu.async_copy(ei_h.at[pl.ds(c * 2 * _E + base, _EPT)],
                             src_v, sem_in),
            pltpu.async_copy(ei_h.at[pl.ds(c * 2 * _E + _E + base, _EPT)],
                             dst_v, sem_in),
            pltpu.async_copy(ssrc_h.at[pl.ds(c * _N, _N)], ssrc_v, sem_in),
            pltpu.async_copy(sdst_h.at[pl.ds(c * _N, _N)], sdst_v, sem_in),
            pltpu.async_copy(esc_h.at[pl.ds(c * _E + base, _EPT)],
                             esc_v, sem_in),
        ]

        @pl.when(s == 0)
        def _():
            pltpu.sync_copy(zeros_h, a_sh)

        for cp in cps:
            cp.wait()

        @pl.loop(0, _EPT // _LANES)
        def _(j):
            si = src_v[pl.ds(j * _LANES, _LANES)]
            di = dst_v[pl.ds(j * _LANES, _LANES)]
            sa = plsc.load_gather(ssrc_v, [si])
            sb = plsc.load_gather(sdst_v, [di])
            t = sa + sb + esc_v[pl.ds(j * _LANES, _LANES)]
            t = jnp.maximum(t, t * _F32(0.2))
            ex_v[j // 8, pl.ds((j % 8) * _LANES, _LANES)] = jnp.exp(t)
            idx_v[j // 8, pl.ds((j % 8) * _LANES, _LANES)] = di * _N + si

        plsc.subcore_barrier()
        d0 = pltpu.async_copy(ex_v.at[0], a_sh.at[idx_v.at[0]], sem_sc,
                              add=True)
        d1 = pltpu.async_copy(ex_v.at[1], a_sh.at[idx_v.at[1]], sem_sc,
                              add=True)
        d0.wait()
        d1.wait()
        plsc.subcore_barrier()

        @pl.when(s == 0)
        def _():
            pltpu.sync_copy(a_sh, a_out.at[pl.ds(c * _N * _N, _N * _N)])

    return k


def _sc_edge(ssrc, sdst, esc, ei, zeros):
    return _sc_edge_kernel()(ssrc, sdst, esc, ei, zeros)


def _mid_body(a, h1, b1_e, b1_g, w2_e, w2_g, as2_e, ad2_e, as2_g, ad2_g,
              h2_o, ssrc_o, sdst_o):
    for b, (b1, w2, a_s, a_d) in enumerate(((b1_e, w2_e, as2_e, ad2_e),
                                            (b1_g, w2_g, as2_g, ad2_g))):
        am = a[b]
        den = jnp.sum(am, axis=1, keepdims=True) + _F32(1e-16)
        m = jnp.dot(am, h1[b], preferred_element_type=_F32) / den
        t = jnp.maximum(m + b1[...].reshape(1, -1), 0.0)
        h2 = jnp.dot(t, w2[...], preferred_element_type=_F32)
        h2_o[b] = h2
        av = jnp.concatenate([a_s[...].reshape(1, -1),
                              a_d[...].reshape(1, -1)], axis=0)
        sd = _t_dot(av, h2)
        ssrc_o[pl.ds(b * _N, _N)] = sd[0]
        sdst_o[pl.ds(b * _N, _N)] = sd[1]


def _tc_mid(*args):
    return pl.pallas_call(
        _mid_body,
        out_shape=[
            jax.ShapeDtypeStruct((2, _N, 128), _F32),
            jax.ShapeDtypeStruct((2 * _N,), _F32),
            jax.ShapeDtypeStruct((2 * _N,), _F32),
        ],
    )(*args)


def _fin_body(a, h2, b2_e, wp_e, bp_e, b2_g, wp_g, bp_g,
              w1a, b1a, w1b, b1b, w2a, b2a, w2b, b2b, out_o):
    feats = []
    for b, (b2, wp, bp) in enumerate(((b2_e, wp_e, bp_e),
                                      (b2_g, wp_g, bp_g))):
        am = a[b]
        den = jnp.sum(am, axis=1, keepdims=True) + _F32(1e-16)
        o = jnp.dot(am, h2[b], preferred_element_type=_F32) / den
        o = o + b2[...].reshape(1, -1)
        feats.append(jnp.dot(o, wp[...], preferred_element_type=_F32)
                     + bp[...].reshape(1, -1))
    z = jnp.concatenate(feats, axis=0)
    t = z + jnp.sum(z, axis=0, keepdims=True)
    t = jnp.maximum(jnp.dot(t, w1a[...], preferred_element_type=_F32)
                    + b1a[...].reshape(1, -1), 0.0)
    t = jnp.dot(t, w1b[...], preferred_element_type=_F32) \
        + b1b[...].reshape(1, -1)
    t = jnp.maximum(t, 0.0)
    t = t + jnp.sum(t, axis=0, keepdims=True)
    t = jnp.maximum(jnp.dot(t, w2a[...], preferred_element_type=_F32)
                    + b2a[...].reshape(1, -1), 0.0)
    out_o[...] = (jnp.dot(t, w2b[...], preferred_element_type=_F32)
                  + b2b[...].reshape(1, -1))


def _tc_fin(*args):
    return pl.pallas_call(
        _fin_body,
        out_shape=jax.ShapeDtypeStruct((2 * _N, 128), _F32),
    )(*args)


def kernel(emg_x, emg_edge_index, emg_edge_attr,
           eeg_x, eeg_edge_index, eeg_edge_attr, params):
    pe = params["emg_gat"]
    pg = params["eeg_gat"]
    gin = params["gin"]

    h1, ssrc1, sdst1, esc1, esc2 = _tc_pre(
        emg_x, pe["W1"], pe["as1"], pe["ad1"], emg_edge_attr.T,
        pe["We1"], pe["ae1"], pe["We2"], pe["ae2"],
        eeg_x, pg["W1"], pg["as1"], pg["ad1"], eeg_edge_attr.T,
        pg["We1"], pg["ae1"], pg["We2"], pg["ae2"])

    zeros = jnp.zeros((_N * _N,), _F32)
    ei = jnp.concatenate([emg_edge_index.reshape(-1),
                          eeg_edge_index.reshape(-1)])
    a1 = _sc_edge(ssrc1, sdst1, esc1, ei, zeros)

    h2, ssrc2, sdst2 = _tc_mid(
        a1.reshape(2, _N, _N), h1, pe["b1"], pg["b1"],
        pe["W2"], pg["W2"], pe["as2"], pe["ad2"], pg["as2"], pg["ad2"])

    a2 = _sc_edge(ssrc2, sdst2, esc2, ei, zeros)

    prj_e = params["emg_proj"]
    prj_g = params["eeg_proj"]
    return _tc_fin(
        a2.reshape(2, _N, _N), h2,
        pe["b2"], prj_e["W"], prj_e["b"],
        pg["b2"], prj_g["W"], prj_g["b"],
        gin["W1a"], gin["b1a"], gin["W1b"], gin["b1b"],
        gin["W2a"], gin["b2a"], gin["W2b"], gin["b2b"])
